# baseline (device time: 24440 ns/iter reference)
import jax
import jax.numpy as jnp
from jax import lax
from jax.experimental import pallas as pl
from jax.experimental.pallas import tpu as pltpu

N_DEV = 4
B = 2
SQ_PER = 128
SKV_PER = 128
SKV = 512
HQ = 4
DH = 64
HD = HQ * DH
BLK = 64
NEG = -1e9


def kernel(x, Wq, K_ext, V_ext, Wo):
    def body(x_ref, wq_ref, k_ref, v_ref, wo_ref, out_ref,
             comm_ref, kg_ref, vg_ref, send_sems, recv_sems):
        my_pos = lax.axis_index("i")
        left = (my_pos - 1) % N_DEV
        right = (my_pos + 1) % N_DEV

        k_own = k_ref[:].reshape(B, SKV_PER, HD).astype(jnp.bfloat16)
        v_own = v_ref[:].reshape(B, SKV_PER, HD).astype(jnp.bfloat16)
        comm_ref[0, 0] = k_own
        comm_ref[0, 1] = v_own
        kg_ref[:, pl.ds(my_pos * SKV_PER, SKV_PER), :] = k_own
        vg_ref[:, pl.ds(my_pos * SKV_PER, SKV_PER), :] = v_own

        barrier_sem = pltpu.get_barrier_semaphore()
        for nbr in [left, right]:
            pl.semaphore_signal(
                barrier_sem, inc=1,
                device_id=(nbr,), device_id_type=pl.DeviceIdType.MESH,
            )
        pl.semaphore_wait(barrier_sem, 2)

        for h in range(N_DEV - 1):
            rdma = pltpu.make_async_remote_copy(
                src_ref=comm_ref.at[h],
                dst_ref=comm_ref.at[h + 1],
                send_sem=send_sems.at[h],
                recv_sem=recv_sems.at[h],
                device_id=(right,),
                device_id_type=pl.DeviceIdType.MESH,
            )
            rdma.start()
            rdma.wait()
            origin = (my_pos - h - 1) % N_DEV
            kg_ref[:, pl.ds(origin * SKV_PER, SKV_PER), :] = comm_ref[h + 1, 0]
            vg_ref[:, pl.ds(origin * SKV_PER, SKV_PER), :] = comm_ref[h + 1, 1]

        row = lax.broadcasted_iota(jnp.int32, (SQ_PER, SKV), 0)
        col = lax.broadcasted_iota(jnp.int32, (SQ_PER, SKV), 1)
        qb = (my_pos * SQ_PER + row) // BLK
        kb = col // BLK
        mask = (qb == kb) | (kb == 0) | ((qb + kb) % 3 == 0)

        wq = wq_ref[:].astype(jnp.bfloat16)
        wo = wo_ref[:].astype(jnp.bfloat16)
        for b in range(B):
            xb = x_ref[b].astype(jnp.bfloat16)
            q_all = lax.dot_general(
                xb, wq, (((1,), (0,)), ((), ())),
                preferred_element_type=jnp.float32,
            ).astype(jnp.bfloat16)
            ctx_parts = []
            for h in range(HQ):
                q = q_all[:, h * DH:(h + 1) * DH]
                k = kg_ref[b, :, h * DH:(h + 1) * DH]
                s = lax.dot_general(
                    q, k, (((1,), (1,)), ((), ())),
                    preferred_element_type=jnp.float32,
                ) * 0.125
                s = jnp.where(mask, s, NEG)
                m = jnp.max(s, axis=1, keepdims=True)
                w = jnp.exp(s - m)
                w = w / jnp.sum(w, axis=1, keepdims=True)
                v = vg_ref[b, :, h * DH:(h + 1) * DH]
                ctx_parts.append(lax.dot_general(
                    w.astype(jnp.bfloat16), v, (((1,), (0,)), ((), ())),
                    preferred_element_type=jnp.float32,
                ))
            ctx = jnp.concatenate(ctx_parts, axis=1).astype(jnp.bfloat16)
            out_ref[b] = lax.dot_general(
                ctx, wo, (((1,), (0,)), ((), ())),
                preferred_element_type=jnp.float32,
            )

    return pl.pallas_call(
        body,
        out_shape=jax.ShapeDtypeStruct((B, SQ_PER, 512), jnp.float32),
        in_specs=[pl.BlockSpec(memory_space=pltpu.VMEM)] * 5,
        out_specs=pl.BlockSpec(memory_space=pltpu.VMEM),
        scratch_shapes=[
            pltpu.VMEM((N_DEV, 2, B, SKV_PER, HD), jnp.bfloat16),
            pltpu.VMEM((B, SKV, HD), jnp.bfloat16),
            pltpu.VMEM((B, SKV, HD), jnp.bfloat16),
            pltpu.SemaphoreType.DMA((N_DEV - 1,)),
            pltpu.SemaphoreType.DMA((N_DEV - 1,)),
        ],
        compiler_params=pltpu.CompilerParams(collective_id=0),
    )(x, Wq, K_ext, V_ext, Wo)


# device time: 18342 ns/iter; 1.3325x vs baseline; 1.3325x over previous
import jax
import jax.numpy as jnp
from jax import lax
from jax.experimental import pallas as pl
from jax.experimental.pallas import tpu as pltpu

N_DEV = 4
B = 2
SQ_PER = 128
SKV_PER = 128
HQ = 4
DH = 64
HD = HQ * DH
BLK = 64
NEG = -1e9


def kernel(x, Wq, K_ext, V_ext, Wo):
    def body(x_ref, wq_ref, k_ref, v_ref, wo_ref, out_ref,
             stage_ref, comm_ref, send_sems, recv_sems):
        my_pos = lax.axis_index("i")
        left = (my_pos - 1) % N_DEV
        right = (my_pos + 1) % N_DEV
        diag = (my_pos + 2) % N_DEV

        k_own = k_ref[:].reshape(B, SKV_PER, HD).astype(jnp.bfloat16)
        v_own = v_ref[:].reshape(B, SKV_PER, HD).astype(jnp.bfloat16)
        stage_ref[0] = k_own
        stage_ref[1] = v_own

        barrier_sem = pltpu.get_barrier_semaphore()
        for nbr in [left, right, diag]:
            pl.semaphore_signal(
                barrier_sem, inc=1,
                device_id=(nbr,), device_id_type=pl.DeviceIdType.MESH,
            )
        pl.semaphore_wait(barrier_sem, 3)

        rdmas = []
        for i, tgt in enumerate([right, left, diag]):
            rdma = pltpu.make_async_remote_copy(
                src_ref=stage_ref,
                dst_ref=comm_ref.at[i],
                send_sem=send_sems.at[i],
                recv_sem=recv_sems.at[i],
                device_id=(tgt,),
                device_id_type=pl.DeviceIdType.MESH,
            )
            rdma.start()
            rdmas.append(rdma)

        wq = wq_ref[:].astype(jnp.bfloat16)
        wo = wo_ref[:].astype(jnp.bfloat16)
        q_all = []
        for b in range(B):
            q_all.append(lax.dot_general(
                x_ref[b].astype(jnp.bfloat16), wq, (((1,), (0,)), ((), ())),
                preferred_element_type=jnp.float32,
            ).astype(jnp.bfloat16))

        row = lax.broadcasted_iota(jnp.int32, (SQ_PER, SKV_PER), 0)
        col = lax.broadcasted_iota(jnp.int32, (SQ_PER, SKV_PER), 1)
        qb = my_pos * 2 + row // BLK

        def chunk_mask(origin):
            kb = origin * 2 + col // BLK
            return (qb == kb) | (kb == 0) | ((qb + kb) % 3 == 0)

        m = [[jnp.full((SQ_PER, 1), -1e30, jnp.float32) for _ in range(HQ)]
             for _ in range(B)]
        l = [[jnp.zeros((SQ_PER, 1), jnp.float32) for _ in range(HQ)]
             for _ in range(B)]
        acc = [[jnp.zeros((SQ_PER, DH), jnp.float32) for _ in range(HQ)]
               for _ in range(B)]

        def consume(origin, k_chunk, v_chunk):
            msk = chunk_mask(origin)
            for b in range(B):
                for h in range(HQ):
                    q = q_all[b][:, h * DH:(h + 1) * DH]
                    kc = k_chunk[b][:, h * DH:(h + 1) * DH]
                    vc = v_chunk[b][:, h * DH:(h + 1) * DH]
                    s = lax.dot_general(
                        q, kc, (((1,), (1,)), ((), ())),
                        preferred_element_type=jnp.float32,
                    ) * 0.125
                    s = jnp.where(msk, s, NEG)
                    m_new = jnp.maximum(m[b][h], jnp.max(s, axis=1, keepdims=True))
                    p = jnp.exp(s - m_new)
                    alpha = jnp.exp(m[b][h] - m_new)
                    l[b][h] = l[b][h] * alpha + jnp.sum(p, axis=1, keepdims=True)
                    acc[b][h] = acc[b][h] * alpha + lax.dot_general(
                        p.astype(jnp.bfloat16), vc, (((1,), (0,)), ((), ())),
                        preferred_element_type=jnp.float32,
                    )
                    m[b][h] = m_new

        consume(my_pos, [k_own[b] for b in range(B)], [v_own[b] for b in range(B)])

        for i, origin in enumerate([left, right, diag]):
            rdmas[i].wait_recv()
            consume(origin,
                    [comm_ref[i, 0, b] for b in range(B)],
                    [comm_ref[i, 1, b] for b in range(B)])

        for b in range(B):
            ctx = jnp.concatenate(
                [acc[b][h] / l[b][h] for h in range(HQ)], axis=1
            ).astype(jnp.bfloat16)
            out_ref[b] = lax.dot_general(
                ctx, wo, (((1,), (0,)), ((), ())),
                preferred_element_type=jnp.float32,
            )

        for rdma in rdmas:
            rdma.wait_send()

    return pl.pallas_call(
        body,
        out_shape=jax.ShapeDtypeStruct((B, SQ_PER, 512), jnp.float32),
        in_specs=[pl.BlockSpec(memory_space=pltpu.VMEM)] * 5,
        out_specs=pl.BlockSpec(memory_space=pltpu.VMEM),
        scratch_shapes=[
            pltpu.VMEM((2, B, SKV_PER, HD), jnp.bfloat16),
            pltpu.VMEM((3, 2, B, SKV_PER, HD), jnp.bfloat16),
            pltpu.SemaphoreType.DMA((3,)),
            pltpu.SemaphoreType.DMA((3,)),
        ],
        compiler_params=pltpu.CompilerParams(collective_id=0),
    )(x, Wq, K_ext, V_ext, Wo)
